# Initial kernel scaffold; baseline (speedup 1.0000x reference)
import functools

import jax
import jax.numpy as jnp
from jax import lax
from jax.experimental import pallas as pl
from jax.experimental.pallas import tpu as pltpu
from jax.experimental.pallas import tpu_sc as plsc

BSZ = 16
SEQ = 2048
HID = 128
NSEG = 512
CHUNK = 256
NCHUNK = SEQ // CHUNK
HGRP = HID // 16


def _tagop_body(values_hbm, scores_hbm, index_hbm, mean_out, max_out,
                idx_v, sc_v, vals_v, acc_v, cnt_v, mx_v, mxf_v):
    c = lax.axis_index("c")
    s = lax.axis_index("s")
    wid = s * 2 + c

    @pl.when(wid < BSZ)
    def _run():
        b = wid
        pltpu.sync_copy(index_hbm.at[b], idx_v)
        pltpu.sync_copy(scores_hbm.at[b], sc_v)

        zero = jnp.zeros((16,), jnp.float32)
        neg = jnp.full((16,), -jnp.inf, jnp.float32)

        def zbody(r, _):
            for h in range(HGRP):
                acc_v[r, pl.ds(h * 16, 16)] = zero
            cnt_v[r] = zero
            mx_v[r] = neg
            return 0

        lax.fori_loop(0, NSEG, zbody, 0)

        def chunk_body(ck, _):
            pltpu.sync_copy(values_hbm.at[b, pl.ds(ck * CHUNK, CHUNK)], vals_v)
            base = ck * CHUNK

            def token_body(t, _):
                i = idx_v[base + t]
                for h in range(HGRP):
                    col = pl.ds(h * 16, 16)
                    acc_v[i, col] = acc_v[i, col] + vals_v[t, col]
                cnt_v[i] = cnt_v[i] + 1.0
                sv = jnp.full((16,), sc_v[base + t], jnp.float32)
                mx_v[i] = jnp.maximum(mx_v[i], sv)
                return 0

            lax.fori_loop(0, CHUNK, token_body, 0)
            return 0

        lax.fori_loop(0, NCHUNK, chunk_body, 0)

        def fbody(r, _):
            recip = 1.0 / jnp.maximum(cnt_v[r], 1.0)
            for h in range(HGRP):
                col = pl.ds(h * 16, 16)
                acc_v[r, col] = acc_v[r, col] * recip
            return 0

        lax.fori_loop(0, NSEG, fbody, 0)

        lanes = lax.iota(jnp.int32, 16)
        zlane = jnp.zeros((16,), jnp.int32)

        def gbody(g, _):
            rows = g * 16 + lanes
            mrow = plsc.load_gather(mx_v, [rows, zlane])
            crow = plsc.load_gather(cnt_v, [rows, zlane])
            mxf_v[pl.ds(g * 16, 16)] = jnp.where(crow > 0.0, mrow, 0.0)
            return 0

        lax.fori_loop(0, NSEG // 16, gbody, 0)

        pltpu.sync_copy(acc_v, mean_out.at[b])
        pltpu.sync_copy(mxf_v, max_out.at[b])


@jax.jit
def _tagop(values, scores, index):
    mesh = plsc.VectorSubcoreMesh(core_axis_name="c", subcore_axis_name="s")
    fn = functools.partial(
        pl.kernel,
        mesh=mesh,
        out_type=(
            jax.ShapeDtypeStruct((BSZ, NSEG, HID), jnp.float32),
            jax.ShapeDtypeStruct((BSZ, NSEG), jnp.float32),
        ),
        scratch_types=[
            pltpu.VMEM((SEQ,), jnp.int32),
            pltpu.VMEM((SEQ,), jnp.float32),
            pltpu.VMEM((CHUNK, HID), jnp.float32),
            pltpu.VMEM((NSEG, HID), jnp.float32),
            pltpu.VMEM((NSEG, 16), jnp.float32),
            pltpu.VMEM((NSEG, 16), jnp.float32),
            pltpu.VMEM((NSEG,), jnp.float32),
        ],
    )(_tagop_body)
    return fn(values, scores, index)


def kernel(values, scores, index):
    return _tagop(values, scores, index)


# speedup vs baseline: 2.1377x; 2.1377x over previous
import functools

import jax
import jax.numpy as jnp
from jax import lax
from jax.experimental import pallas as pl
from jax.experimental.pallas import tpu as pltpu
from jax.experimental.pallas import tpu_sc as plsc

BSZ = 16
SEQ = 2048
HID = 128
NSEG = 512
CHUNK = 128
NCHUNK = SEQ // CHUNK
HGRP = HID // 16


def _tagop_body(values_hbm, scores_hbm, index_hbm, mean_out, max_out,
                idx_v, sc_v, vals_v, acc_v, cnt_v, mx_v, mxf_v):
    c = lax.axis_index("c")
    s = lax.axis_index("s")
    wid = s * 2 + c

    @pl.when(wid < BSZ)
    def _run():
        b = wid
        pltpu.sync_copy(index_hbm.at[b], idx_v)
        pltpu.sync_copy(scores_hbm.at[b], sc_v)

        zero = jnp.zeros((16,), jnp.float32)
        neg = jnp.full((16,), -jnp.inf, jnp.float32)

        def zbody(r, _):
            for h in range(HGRP):
                acc_v[r, pl.ds(h * 16, 16)] = zero
            cnt_v[pl.ds(r * 16, 16)] = zero
            mx_v[pl.ds(r * 16, 16)] = neg
            return 0

        lax.fori_loop(0, NSEG, zbody, 0)

        def chunk_body(ck, _):
            pltpu.sync_copy(values_hbm.at[b, pl.ds(ck * CHUNK, CHUNK)], vals_v)
            base = ck * CHUNK

            def grp_body(g, _):
                gb = base + g * 16
                iv = idx_v[pl.ds(gb, 16)]
                sv16 = sc_v[pl.ds(gb, 16)]
                tb = g * 16
                for j in range(16):
                    i = iv[j]
                    for h in range(HGRP):
                        col = pl.ds(h * 16, 16)
                        acc_v[i, col] = acc_v[i, col] + vals_v[tb + j, col]
                    ci = pl.ds(i * 16, 16)
                    cnt_v[ci] = cnt_v[ci] + 1.0
                    sv = jnp.full((16,), sv16[j], jnp.float32)
                    mx_v[ci] = jnp.maximum(mx_v[ci], sv)
                return 0

            lax.fori_loop(0, CHUNK // 16, grp_body, 0)
            return 0

        lax.fori_loop(0, NCHUNK, chunk_body, 0)

        def fbody(r, _):
            recip = 1.0 / jnp.maximum(cnt_v[pl.ds(r * 16, 16)], 1.0)
            for h in range(HGRP):
                col = pl.ds(h * 16, 16)
                acc_v[r, col] = acc_v[r, col] * recip
            return 0

        lax.fori_loop(0, NSEG, fbody, 0)

        lanes = lax.iota(jnp.int32, 16)

        def gbody(g, _):
            m = jnp.zeros((16,), jnp.float32)
            cz = jnp.zeros((16,), jnp.float32)
            for j in range(16):
                r = g * 16 + j
                sel = lanes == j
                m = jnp.where(sel, mx_v[pl.ds(r * 16, 16)], m)
                cz = jnp.where(sel, cnt_v[pl.ds(r * 16, 16)], cz)
            mxf_v[pl.ds(g * 16, 16)] = jnp.where(cz > 0.0, m, 0.0)
            return 0

        lax.fori_loop(0, NSEG // 16, gbody, 0)

        pltpu.sync_copy(acc_v, mean_out.at[b])
        pltpu.sync_copy(mxf_v, max_out.at[b])


@jax.jit
def _tagop(values, scores, index):
    mesh = plsc.VectorSubcoreMesh(core_axis_name="c", subcore_axis_name="s")
    fn = functools.partial(
        pl.kernel,
        mesh=mesh,
        out_type=(
            jax.ShapeDtypeStruct((BSZ, NSEG, HID), jnp.float32),
            jax.ShapeDtypeStruct((BSZ, NSEG), jnp.float32),
        ),
        scratch_types=[
            pltpu.VMEM((SEQ,), jnp.int32),
            pltpu.VMEM((SEQ,), jnp.float32),
            pltpu.VMEM((CHUNK, HID), jnp.float32),
            pltpu.VMEM((NSEG, HID), jnp.float32),
            pltpu.VMEM((NSEG * 16,), jnp.float32),
            pltpu.VMEM((NSEG * 16,), jnp.float32),
            pltpu.VMEM((NSEG,), jnp.float32),
        ],
    )(_tagop_body)
    return fn(values, scores, index)


def kernel(values, scores, index):
    return _tagop(values, scores, index)


# speedup vs baseline: 2.5465x; 1.1913x over previous
import functools

import jax
import jax.numpy as jnp
from jax import lax
from jax.experimental import pallas as pl
from jax.experimental.pallas import tpu as pltpu
from jax.experimental.pallas import tpu_sc as plsc

BSZ = 16
SEQ = 2048
HID = 128
NSEG = 512
CHUNK = 128
NCHUNK = SEQ // CHUNK
HGRP = HID // 16


def _tagop_body(values_hbm, scores_hbm, index_hbm, mean_out, max_out,
                idx_v, sc_v, vals_v, acc_v, cnt_v, mx_v, mx2_v, mxf_v):
    c = lax.axis_index("c")
    s = lax.axis_index("s")
    wid = s * 2 + c

    @pl.when(wid < BSZ)
    def _run():
        b = wid
        pltpu.sync_copy(index_hbm.at[b], idx_v)
        pltpu.sync_copy(scores_hbm.at[b], sc_v)

        zero = jnp.zeros((16,), jnp.float32)
        neg = jnp.full((16,), -jnp.inf, jnp.float32)

        def zbody(r, _):
            for h in range(HGRP):
                acc_v[r, pl.ds(h * 16, 16)] = zero
            cnt_v[pl.ds(r * 16, 16)] = zero
            mx_v[pl.ds(r * 16, 16)] = neg
            mx2_v[pl.ds(r * 16, 16)] = neg
            return 0

        lax.fori_loop(0, NSEG, zbody, 0)

        def chunk_body(ck, _):
            pltpu.sync_copy(values_hbm.at[b, pl.ds(ck * CHUNK, CHUNK)], vals_v)
            base = ck * CHUNK

            def grp_body(g, _):
                gb = base + g * 16
                iv = idx_v[pl.ds(gb, 16)]
                sv16 = sc_v[pl.ds(gb, 16)]
                tb = g * 16
                one = jnp.ones((16,), jnp.float32)
                for j in range(16):
                    i = iv[j]
                    for h in range(HGRP):
                        col = pl.ds(h * 16, 16)
                        plsc.addupdate(acc_v.at[i, col], vals_v[tb + j, col])
                    ci = pl.ds(i * 16, 16)
                    plsc.addupdate(cnt_v.at[ci], one)
                    sv = jnp.full((16,), sv16[j], jnp.float32)
                    mref = mx_v if j % 2 == 0 else mx2_v
                    mref[ci] = jnp.maximum(mref[ci], sv)
                return 0

            lax.fori_loop(0, CHUNK // 16, grp_body, 0)
            return 0

        lax.fori_loop(0, NCHUNK, chunk_body, 0)

        def fbody(r, _):
            recip = 1.0 / jnp.maximum(cnt_v[pl.ds(r * 16, 16)], 1.0)
            for h in range(HGRP):
                col = pl.ds(h * 16, 16)
                acc_v[r, col] = acc_v[r, col] * recip
            return 0

        lax.fori_loop(0, NSEG, fbody, 0)

        lanes = lax.iota(jnp.int32, 16)

        def gbody(g, _):
            m = jnp.zeros((16,), jnp.float32)
            cz = jnp.zeros((16,), jnp.float32)
            for j in range(16):
                r = g * 16 + j
                sel = lanes == j
                rr = pl.ds(r * 16, 16)
                m = jnp.where(sel, jnp.maximum(mx_v[rr], mx2_v[rr]), m)
                cz = jnp.where(sel, cnt_v[rr], cz)
            mxf_v[pl.ds(g * 16, 16)] = jnp.where(cz > 0.0, m, 0.0)
            return 0

        lax.fori_loop(0, NSEG // 16, gbody, 0)

        pltpu.sync_copy(acc_v, mean_out.at[b])
        pltpu.sync_copy(mxf_v, max_out.at[b])


@jax.jit
def _tagop(values, scores, index):
    mesh = plsc.VectorSubcoreMesh(core_axis_name="c", subcore_axis_name="s")
    fn = functools.partial(
        pl.kernel,
        mesh=mesh,
        out_type=(
            jax.ShapeDtypeStruct((BSZ, NSEG, HID), jnp.float32),
            jax.ShapeDtypeStruct((BSZ, NSEG), jnp.float32),
        ),
        scratch_types=[
            pltpu.VMEM((SEQ,), jnp.int32),
            pltpu.VMEM((SEQ,), jnp.float32),
            pltpu.VMEM((CHUNK, HID), jnp.float32),
            pltpu.VMEM((NSEG, HID), jnp.float32),
            pltpu.VMEM((NSEG * 16,), jnp.float32),
            pltpu.VMEM((NSEG * 16,), jnp.float32),
            pltpu.VMEM((NSEG * 16,), jnp.float32),
            pltpu.VMEM((NSEG,), jnp.float32),
        ],
    )(_tagop_body)
    return fn(values, scores, index)


def kernel(values, scores, index):
    return _tagop(values, scores, index)


# speedup vs baseline: 3.6882x; 1.4483x over previous
import functools

import jax
import jax.numpy as jnp
from jax import lax
from jax.experimental import pallas as pl
from jax.experimental.pallas import tpu as pltpu
from jax.experimental.pallas import tpu_sc as plsc

BSZ = 16
SEQ = 2048
HID = 128
NSEG = 512
CHUNK = 128
NCHUNK = SEQ // CHUNK
HGRP = HID // 16


def _tagop_body(values_hbm, scores_hbm, index_hbm, mean_out, max_out,
                idx_v, sc_v, vals_v, acc_v, cnt_v, mx_v, mx2_v, mxf_v):
    c = lax.axis_index("c")
    s = lax.axis_index("s")
    wid = s * 2 + c

    @pl.when(wid < BSZ)
    def _run():
        b = wid
        pltpu.sync_copy(index_hbm.at[b], idx_v)
        pltpu.sync_copy(scores_hbm.at[b], sc_v)

        zero = jnp.zeros((16,), jnp.float32)
        neg = jnp.full((16,), -jnp.inf, jnp.float32)

        def zbody(r, _):
            for h in range(HGRP):
                acc_v[r, pl.ds(h * 16, 16)] = zero
            cnt_v[pl.ds(r * 16, 16)] = zero
            mx_v[pl.ds(r * 16, 16)] = neg
            mx2_v[pl.ds(r * 16, 16)] = neg
            return 0

        lax.fori_loop(0, NSEG, zbody, 0)

        def chunk_body(ck, _):
            pltpu.sync_copy(values_hbm.at[b, pl.ds(ck * CHUNK, CHUNK)], vals_v)
            base = ck * CHUNK

            def grp_body(g, _):
                gb = base + g * 16
                iv = idx_v[pl.ds(gb, 16)]
                sv16 = sc_v[pl.ds(gb, 16)]
                tb = g * 16
                one = jnp.ones((16,), jnp.float32)

                def ldrow(j):
                    return [vals_v[tb + j, pl.ds(h * 16, 16)]
                            for h in range(HGRP)]

                vrow = ldrow(0)
                for j in range(16):
                    nxt = ldrow(j + 1) if j < 15 else None
                    i = iv[j]
                    for h in range(HGRP):
                        plsc.addupdate(acc_v.at[i, pl.ds(h * 16, 16)], vrow[h])
                    ci = pl.ds(i * 16, 16)
                    plsc.addupdate(cnt_v.at[ci], one)
                    sv = jnp.full((16,), sv16[j], jnp.float32)
                    mref = mx_v if j % 2 == 0 else mx2_v
                    mref[ci] = jnp.maximum(mref[ci], sv)
                    vrow = nxt
                return 0

            lax.fori_loop(0, CHUNK // 16, grp_body, 0)
            return 0

        lax.fori_loop(0, NCHUNK, chunk_body, 0)

        def fbody(r, _):
            recip = 1.0 / jnp.maximum(cnt_v[pl.ds(r * 16, 16)], 1.0)
            for h in range(HGRP):
                col = pl.ds(h * 16, 16)
                acc_v[r, col] = acc_v[r, col] * recip
            return 0

        lax.fori_loop(0, NSEG, fbody, 0)

        lanes = lax.iota(jnp.int32, 16)

        def gbody(g, _):
            m = jnp.zeros((16,), jnp.float32)
            cz = jnp.zeros((16,), jnp.float32)
            for j in range(16):
                r = g * 16 + j
                sel = lanes == j
                rr = pl.ds(r * 16, 16)
                m = jnp.where(sel, jnp.maximum(mx_v[rr], mx2_v[rr]), m)
                cz = jnp.where(sel, cnt_v[rr], cz)
            mxf_v[pl.ds(g * 16, 16)] = jnp.where(cz > 0.0, m, 0.0)
            return 0

        lax.fori_loop(0, NSEG // 16, gbody, 0)

        pltpu.sync_copy(acc_v, mean_out.at[b])
        pltpu.sync_copy(mxf_v, max_out.at[b])


@jax.jit
def _tagop(values, scores, index):
    mesh = plsc.VectorSubcoreMesh(core_axis_name="c", subcore_axis_name="s")
    fn = functools.partial(
        pl.kernel,
        mesh=mesh,
        out_type=(
            jax.ShapeDtypeStruct((BSZ, NSEG, HID), jnp.float32),
            jax.ShapeDtypeStruct((BSZ, NSEG), jnp.float32),
        ),
        scratch_types=[
            pltpu.VMEM((SEQ,), jnp.int32),
            pltpu.VMEM((SEQ,), jnp.float32),
            pltpu.VMEM((CHUNK, HID), jnp.float32),
            pltpu.VMEM((NSEG, HID), jnp.float32),
            pltpu.VMEM((NSEG * 16,), jnp.float32),
            pltpu.VMEM((NSEG * 16,), jnp.float32),
            pltpu.VMEM((NSEG * 16,), jnp.float32),
            pltpu.VMEM((NSEG,), jnp.float32),
        ],
    )(_tagop_body)
    return fn(values, scores, index)


def kernel(values, scores, index):
    return _tagop(values, scores, index)
